# baseline (speedup 1.0000x reference)
import functools

import jax
import jax.numpy as jnp
from jax import lax
from jax.experimental import pallas as pl
from jax.experimental.pallas import tpu as pltpu
from jax.experimental.pallas import tpu_sc as plsc

D = 16
B = 262144
V = 2146689
NC = 2
NS = 16
L = 16
NW = NC * NS
BPW = B // NW
C = 2048
NCHUNK = BPW // C

S1 = 129
S2 = 129 * 129


def kernel(idx, table):
    i0 = idx[:, 0]
    i1 = idx[:, 1]
    i2 = idx[:, 2]
    table_t = table.T
    mesh = plsc.VectorSubcoreMesh(core_axis_name="c", subcore_axis_name="s")

    @functools.partial(
        pl.kernel,
        mesh=mesh,
        out_type=jax.ShapeDtypeStruct((D, B), jnp.float32),
        compiler_params=pltpu.CompilerParams(
            needs_layout_passes=False, use_tc_tiling_on_sc=False
        ),
        scratch_types=[
            pltpu.VMEM((C,), jnp.int32),
            pltpu.VMEM((C,), jnp.int32),
            pltpu.VMEM((C,), jnp.int32),
            pltpu.VMEM((C,), jnp.int32),
            pltpu.VMEM((D, C), jnp.float32),
            pltpu.SemaphoreType.DMA,
            pltpu.SemaphoreType.DMA,
        ],
    )
    def grid_gather(i0_hbm, i1_hbm, i2_hbm, tab_hbm, out_hbm,
                    a_v, b_v, c_v, lin_v, planes_v, gsem, wsem):
        wid = lax.axis_index("s") * NC + lax.axis_index("c")
        base_w = wid * BPW

        def chunk_body(ci, carry):
            base = base_w + ci * C
            pltpu.sync_copy(i0_hbm.at[pl.ds(base, C)], a_v)
            pltpu.sync_copy(i1_hbm.at[pl.ds(base, C)], b_v)
            pltpu.sync_copy(i2_hbm.at[pl.ds(base, C)], c_v)

            def lin_body(i, carry2):
                s = pl.ds(i * L, L)
                lin_v[s] = a_v[s] + S1 * b_v[s] + S2 * c_v[s]
                return carry2

            lax.fori_loop(0, C // L, lin_body, 0)

            gcopies = []
            for f in range(D):
                gcopies.append(pltpu.async_copy(
                    tab_hbm.at[f].at[lin_v], planes_v.at[f], gsem))
            for cp in gcopies:
                cp.wait()

            wcopies = []
            for f in range(D):
                wcopies.append(pltpu.async_copy(
                    planes_v.at[f], out_hbm.at[f, pl.ds(base, C)], wsem))
            for cp in wcopies:
                cp.wait()
            return carry

        lax.fori_loop(0, NCHUNK, chunk_body, 0)

    out_t = grid_gather(i0, i1, i2, table_t)
    return out_t.T


# speedup vs baseline: 3.3086x; 3.3086x over previous
import functools

import jax
import jax.numpy as jnp
from jax import lax
from jax.experimental import pallas as pl
from jax.experimental.pallas import tpu as pltpu
from jax.experimental.pallas import tpu_sc as plsc

D = 16
B = 262144
V = 2146689
NC = 2
NS = 16
L = 16
NW = NC * NS
BPW = B // NW
C = 2048
NCHUNK = BPW // C

S1 = 129
S2 = 129 * 129


def kernel(idx, table):
    i0 = idx[:, 0]
    i1 = idx[:, 1]
    i2 = idx[:, 2]
    mesh = plsc.VectorSubcoreMesh(core_axis_name="c", subcore_axis_name="s")

    @functools.partial(
        pl.kernel,
        mesh=mesh,
        out_type=jax.ShapeDtypeStruct((D, B), jnp.float32),
        compiler_params=pltpu.CompilerParams(
            needs_layout_passes=False, use_tc_tiling_on_sc=False
        ),
        scratch_types=[
            pltpu.VMEM((C,), jnp.int32),
            pltpu.VMEM((C,), jnp.int32),
            pltpu.VMEM((C,), jnp.int32),
            pltpu.VMEM((C,), jnp.int32),
            pltpu.VMEM((C,), jnp.int32),
            pltpu.VMEM((C, D), jnp.float32),
            pltpu.VMEM((C, D), jnp.float32),
            pltpu.VMEM((D, C), jnp.float32),
            pltpu.SemaphoreType.DMA,
            pltpu.SemaphoreType.DMA,
        ],
    )
    def grid_gather(i0_hbm, i1_hbm, i2_hbm, tab_hbm, out_hbm,
                    a_v, b_v, c_v, lin0_v, lin1_v, rows0_v, rows1_v,
                    planes_v, gsem, wsem):
        wid = lax.axis_index("s") * NC + lax.axis_index("c")
        base_w = wid * BPW
        lin_bufs = (lin0_v, lin1_v)
        row_bufs = (rows0_v, rows1_v)
        lane = lax.iota(jnp.int32, L)
        cols = [jnp.full((L,), f, jnp.int32) for f in range(D)]

        def fire_gather(ci):
            base = base_w + ci * C
            lin_v = lin_bufs[ci % 2]
            pltpu.sync_copy(i0_hbm.at[pl.ds(base, C)], a_v)
            pltpu.sync_copy(i1_hbm.at[pl.ds(base, C)], b_v)
            pltpu.sync_copy(i2_hbm.at[pl.ds(base, C)], c_v)

            def lin_body(i, carry2):
                s = pl.ds(i * L, L)
                lin_v[s] = a_v[s] + S1 * b_v[s] + S2 * c_v[s]
                return carry2

            lax.fori_loop(0, C // L, lin_body, 0)
            return pltpu.async_copy(tab_hbm.at[lin_v], row_bufs[ci % 2], gsem)

        pending = fire_gather(0)
        wcopies = []
        for ci in range(NCHUNK):
            base = base_w + ci * C
            rows_v = row_bufs[ci % 2]
            pending.wait()
            if ci + 1 < NCHUNK:
                pending = fire_gather(ci + 1)
            for cp in wcopies:
                cp.wait()
            wcopies = []

            def tr_body(j, carry2):
                row_idx = j * L + lane
                for f in range(D):
                    col = plsc.load_gather(rows_v, [row_idx, cols[f]])
                    planes_v[f, pl.ds(j * L, L)] = col
                return carry2

            lax.fori_loop(0, C // L, tr_body, 0)
            for f in range(D):
                wcopies.append(pltpu.async_copy(
                    planes_v.at[f], out_hbm.at[f, pl.ds(base, C)], wsem))
        for cp in wcopies:
            cp.wait()

    out_t = grid_gather(i0, i1, i2, table)
    return out_t.T


# speedup vs baseline: 7.0920x; 2.1435x over previous
import functools

import jax
import jax.numpy as jnp
from jax import lax
from jax.experimental import pallas as pl
from jax.experimental.pallas import tpu as pltpu
from jax.experimental.pallas import tpu_sc as plsc

D = 16
B = 262144
V = 2146689
NC = 2
NS = 16
L = 16
NW = NC * NS
BPW = B // NW
C = 2048
NCHUNK = BPW // C

S1 = 129
S2 = 129 * 129

CH = 1792
NFULL = V // CH
TAIL0 = NFULL * CH
TAILN = (V - TAIL0) // 128 * 128
TAIL1 = TAIL0 + TAILN
CPT = -(-NFULL // NW)


def _transpose_chunk(tab_v, flat_v, n, lane16):
    def blk(jb, carry):
        l0 = jb * L
        for f in range(D):
            vals = tab_v[f, pl.ds(l0, L)]
            plsc.store_scatter(flat_v, [lane16 + (l0 * D + f)], vals)
        return carry

    lax.fori_loop(0, n // L, blk, 0)


def kernel(idx, table):
    i0 = idx[:, 0]
    i1 = idx[:, 1]
    i2 = idx[:, 2]
    table_t = table.T
    mesh = plsc.VectorSubcoreMesh(core_axis_name="c", subcore_axis_name="s")

    @functools.partial(
        pl.kernel,
        mesh=mesh,
        out_type=jax.ShapeDtypeStruct((V * D,), jnp.float32),
        compiler_params=pltpu.CompilerParams(
            needs_layout_passes=False, use_tc_tiling_on_sc=True
        ),
        scratch_types=[
            pltpu.VMEM((D, CH), jnp.float32),
            pltpu.VMEM((D, CH), jnp.float32),
            pltpu.VMEM((CH * D,), jnp.float32),
            pltpu.VMEM((CH * D,), jnp.float32),
            pltpu.SemaphoreType.DMA,
            pltpu.SemaphoreType.DMA,
        ],
    )
    def table_to_rowmajor(tab_hbm, flat_hbm, in0_v, in1_v, out0_v, out1_v,
                          isem, osem):
        wid = lax.axis_index("s") * NC + lax.axis_index("c")
        in_bufs = (in0_v, in1_v)
        out_bufs = (out0_v, out1_v)
        lane16 = lax.iota(jnp.int32, L) * D

        def chunk_of(i):
            return jnp.minimum(wid * CPT + i, NFULL - 1)

        def fire_in(i):
            off = pl.multiple_of(chunk_of(i) * CH, 128)
            return pltpu.async_copy(
                tab_hbm.at[:, pl.ds(off, CH)], in_bufs[i % 2], isem)

        pending = fire_in(0)
        wcopies = []
        for i in range(CPT):
            pending.wait()
            if i + 1 < CPT:
                pending = fire_in(i + 1)
            for cp in wcopies:
                cp.wait()
            wcopies = []
            _transpose_chunk(in_bufs[i % 2], out_bufs[i % 2], CH, lane16)
            woff = pl.multiple_of(chunk_of(i) * (CH * D), 128)
            wcopies.append(pltpu.async_copy(
                out_bufs[i % 2], flat_hbm.at[pl.ds(woff, CH * D)], osem))
        for cp in wcopies:
            cp.wait()

        @pl.when(wid == 0)
        def _tail():
            pltpu.sync_copy(tab_hbm.at[:, pl.ds(TAIL0, TAILN)],
                            in0_v.at[:, pl.ds(0, TAILN)])
            _transpose_chunk(in0_v, out0_v, TAILN, lane16)
            pltpu.sync_copy(out0_v.at[pl.ds(0, TAILN * D)],
                            flat_hbm.at[pl.ds(TAIL0 * D, TAILN * D)])

    @functools.partial(
        pl.kernel,
        mesh=mesh,
        out_type=jax.ShapeDtypeStruct((D, B), jnp.float32),
        compiler_params=pltpu.CompilerParams(
            needs_layout_passes=False, use_tc_tiling_on_sc=False
        ),
        scratch_types=[
            pltpu.VMEM((C,), jnp.int32),
            pltpu.VMEM((C,), jnp.int32),
            pltpu.VMEM((C,), jnp.int32),
            pltpu.VMEM((C,), jnp.int32),
            pltpu.VMEM((C,), jnp.int32),
            pltpu.VMEM((C,), jnp.int32),
            pltpu.VMEM((C,), jnp.int32),
            pltpu.VMEM((C, D), jnp.float32),
            pltpu.VMEM((C, D), jnp.float32),
            pltpu.VMEM((D, C), jnp.float32),
            pltpu.VMEM((L,), jnp.float32),
            pltpu.SemaphoreType.DMA,
            pltpu.SemaphoreType.DMA,
        ],
    )
    def grid_gather(i0_hbm, i1_hbm, i2_hbm, tab_hbm, last_hbm, out_hbm,
                    a_v, b_v, c_v, lin0_v, lin1_v, linr0_v, linr1_v,
                    rows0_v, rows1_v, planes_v, rl_v, gsem, wsem):
        wid = lax.axis_index("s") * NC + lax.axis_index("c")
        base_w = wid * BPW
        lin_bufs = (lin0_v, lin1_v)
        linr_bufs = (linr0_v, linr1_v)
        row_bufs = (rows0_v, rows1_v)
        lane = lax.iota(jnp.int32, L)
        cols = [jnp.full((L,), f, jnp.int32) for f in range(D)]
        pltpu.sync_copy(last_hbm, rl_v)
        rl_b = [plsc.load_gather(rl_v, [cols[f]]) for f in range(D)]

        def fire_gather(ci):
            base = base_w + ci * C
            lin_v = lin_bufs[ci % 2]
            linr_v = linr_bufs[ci % 2]
            pltpu.sync_copy(i0_hbm.at[pl.ds(base, C)], a_v)
            pltpu.sync_copy(i1_hbm.at[pl.ds(base, C)], b_v)
            pltpu.sync_copy(i2_hbm.at[pl.ds(base, C)], c_v)

            def lin_body(i, carry2):
                s = pl.ds(i * L, L)
                raw = a_v[s] + S1 * b_v[s] + S2 * c_v[s]
                linr_v[s] = raw
                lin_v[s] = jnp.minimum(raw, V - 2)
                return carry2

            lax.fori_loop(0, C // L, lin_body, 0)
            return pltpu.async_copy(tab_hbm.at[lin_v], row_bufs[ci % 2], gsem)

        pending = fire_gather(0)
        wcopies = []
        for ci in range(NCHUNK):
            base = base_w + ci * C
            rows_v = row_bufs[ci % 2]
            linr_v = linr_bufs[ci % 2]
            pending.wait()
            if ci + 1 < NCHUNK:
                pending = fire_gather(ci + 1)
            for cp in wcopies:
                cp.wait()
            wcopies = []

            def tr_body(j, carry2):
                s = pl.ds(j * L, L)
                row_idx = j * L + lane
                is_last = linr_v[s] == V - 1
                for f in range(D):
                    col = plsc.load_gather(rows_v, [row_idx, cols[f]])
                    planes_v[f, s] = jnp.where(is_last, rl_b[f], col)
                return carry2

            lax.fori_loop(0, C // L, tr_body, 0)
            for f in range(D):
                wcopies.append(pltpu.async_copy(
                    planes_v.at[f], out_hbm.at[f, pl.ds(base, C)], wsem))
        for cp in wcopies:
            cp.wait()

    tab_flat = table_to_rowmajor(table_t)
    tab_lin = tab_flat.reshape(V, D)
    out_t = grid_gather(i0, i1, i2, tab_lin, table[V - 1])
    return out_t.T
